# baseline (speedup 1.0000x reference)
import functools

import jax
import jax.numpy as jnp
from jax import lax
from jax.experimental import pallas as pl
from jax.experimental.pallas import tpu as pltpu
from jax.experimental.pallas import tpu_sc as plsc


def _sc_gather(table, idx):
  V, D = table.shape
  B = idx.shape[0]
  info = plsc.get_sparse_core_info()
  NC, NS = info.num_cores, info.num_subcores
  NW = NC * NS
  assert B % NW == 0 and (B // NW) % 8 == 0
  b_per_w = B // NW

  mesh = plsc.VectorSubcoreMesh(core_axis_name="c", subcore_axis_name="s")

  @functools.partial(
      pl.kernel,
      mesh=mesh,
      out_type=jax.ShapeDtypeStruct((B, D), jnp.float32),
      scratch_types=[
          pltpu.VMEM((b_per_w,), jnp.int32),
          pltpu.VMEM((b_per_w, D), jnp.float32),
          pltpu.SemaphoreType.DMA,
      ],
      compiler_params=pltpu.CompilerParams(use_tc_tiling_on_sc=False),
  )
  def gather_kernel(table_hbm, idx_hbm, out_hbm, idx_v, rows_v, sem):
    wid = lax.axis_index("s") * NC + lax.axis_index("c")
    base = wid * b_per_w
    pltpu.sync_copy(idx_hbm.at[pl.ds(base, b_per_w)], idx_v)
    pltpu.async_copy(table_hbm.at[idx_v], rows_v, sem).wait()
    pltpu.sync_copy(rows_v, out_hbm.at[pl.ds(base, b_per_w)])

  return gather_kernel(table, idx)


_V_TILE = 2048


def _proj_body(emb_ref, w_ref, b_ref, out_ref):
  acc = jax.lax.dot_general(
      emb_ref[...],
      w_ref[...],
      dimension_numbers=(((1,), (1,)), ((), ())),
      preferred_element_type=jnp.float32,
  )
  out_ref[...] = acc + b_ref[...]


def _tc_project(emb, W, b2d):
  B, E = emb.shape
  V = W.shape[0]
  n_tiles = pl.cdiv(V, _V_TILE)
  return pl.pallas_call(
      _proj_body,
      grid=(n_tiles,),
      in_specs=[
          pl.BlockSpec((B, E), lambda i: (0, 0)),
          pl.BlockSpec((_V_TILE, E), lambda i: (i, 0)),
          pl.BlockSpec((1, _V_TILE), lambda i: (0, i)),
      ],
      out_specs=pl.BlockSpec((B, _V_TILE), lambda i: (0, i)),
      out_shape=jax.ShapeDtypeStruct((B, V), jnp.float32),
  )(emb, W, b2d)


def kernel(center_word, emb_table, W, b):
  idx = center_word.astype(jnp.int32)
  emb = _sc_gather(emb_table, idx)
  return _tc_project(emb, W, b.reshape(1, -1))
